# baseline (speedup 1.0000x reference)
import functools

import jax
import jax.numpy as jnp
from jax import lax
from jax.experimental import pallas as pl
from jax.experimental.pallas import tpu as pltpu
from jax.experimental.pallas import tpu_sc as plsc

_N, _E, _D = 10000, 320000, 128
_NP = 10240
_NC, _NS = 2, 16
_NW = _NC * _NS
_EPW = _E // _NW
_IC = 96
_IR = 108
_KM = _IC
_CHM = _IR
_RPT = _NP // _NS
_ZR = 80
_NZ = _RPT // _ZR
_DW = 8

_mesh = plsc.VectorSubcoreMesh(core_axis_name="c", subcore_axis_name="s")
_sc_params = pltpu.CompilerParams(use_tc_tiling_on_sc=False)


def _fill_const(buf, rows, cols, val):
    ncol = cols // 16

    def body(i, carry):
        r = i // ncol
        c = i % ncol
        buf[r, pl.ds(c * 16, 16)] = jnp.full((16,), val, jnp.float32)
        return carry

    lax.fori_loop(0, rows * ncol, body, 0)


@functools.partial(
    pl.kernel,
    out_type=jax.ShapeDtypeStruct((_NC, _NP, 16), jnp.float32),
    mesh=_mesh,
    scratch_types=[
        pltpu.VMEM((_IR, _IC), jnp.int32),
        pltpu.VMEM((_IC, 16), jnp.float32),
        pltpu.VMEM((_RPT, 16), jnp.float32),
        pltpu.VMEM_SHARED((_NP, 16), jnp.float32),
        pltpu.SemaphoreType.DMA,
    ],
    compiler_params=_sc_params,
)
def _deg_kernel(dst_hbm, deg_out, didx_v, ones_v, buf_v, deg_sp, sem):
    cid = lax.axis_index("c")
    sid = lax.axis_index("s")
    wid = sid * _NC + cid

    pltpu.sync_copy(dst_hbm.at[wid], didx_v)
    _fill_const(ones_v, _IC, 16, 1.0)
    _fill_const(buf_v, _RPT, 16, 0.0)
    pltpu.sync_copy(buf_v, deg_sp.at[pl.ds(sid * _RPT, _RPT)])
    plsc.subcore_barrier()

    def start(i):
        pltpu.async_copy(ones_v, deg_sp.at[didx_v.at[i]], sem, add=True)

    def wait():
        pltpu.make_async_copy(ones_v, deg_sp.at[didx_v.at[0]], sem).wait()

    for k in range(_DW):
        start(k)

    def body(i, carry):
        wait()
        start(i + _DW)
        return carry

    lax.fori_loop(0, _IR - _DW, body, 0)
    for k in range(_DW):
        wait()
    plsc.subcore_barrier()
    pltpu.sync_copy(deg_sp.at[pl.ds(sid * _RPT, _RPT)], buf_v)
    pltpu.sync_copy(buf_v, deg_out.at[cid, pl.ds(sid * _RPT, _RPT)])


@functools.partial(
    pl.kernel,
    out_type=jax.ShapeDtypeStruct((_NC, _NP, _D), jnp.float32),
    mesh=_mesh,
    scratch_types=[
        pltpu.VMEM((_IR, _IC), jnp.int32),
        pltpu.VMEM((_IR, _IC), jnp.int32),
        pltpu.VMEM((_KM, _D), jnp.float32),
        pltpu.VMEM((_KM, _D), jnp.float32),
        pltpu.VMEM_SHARED((_NP, _D), jnp.float32),
        pltpu.SemaphoreType.DMA,
        pltpu.SemaphoreType.DMA,
        pltpu.SemaphoreType.DMA,
        pltpu.SemaphoreType.DMA,
    ],
    compiler_params=_sc_params,
)
def _msg_kernel(y_hbm, src_hbm, dst_hbm, acc_out, sidx_v, didx_v, buf_a, buf_b,
                acc_sp, gs_a, gs_b, ss_a, ss_b):
    cid = lax.axis_index("c")
    sid = lax.axis_index("s")
    wid = sid * _NC + cid

    pltpu.sync_copy(src_hbm.at[wid], sidx_v)
    pltpu.sync_copy(dst_hbm.at[wid], didx_v)
    _fill_const(buf_a, _ZR, _D, 0.0)
    for t in range(_NZ):
        pltpu.sync_copy(buf_a.at[pl.ds(0, _ZR)], acc_sp.at[pl.ds(sid * _RPT + t * _ZR, _ZR)])
    plsc.subcore_barrier()

    def start_g(i, buf, sem):
        pltpu.async_copy(y_hbm.at[sidx_v.at[i]], buf, sem)

    def wait_g(buf, sem):
        pltpu.make_async_copy(y_hbm.at[sidx_v.at[0]], buf, sem).wait()

    def start_s(i, buf, sem):
        pltpu.async_copy(buf, acc_sp.at[didx_v.at[i]], sem, add=True)

    def wait_s(buf, sem):
        pltpu.make_async_copy(buf, acc_sp.at[didx_v.at[0]], sem).wait()

    start_g(0, buf_a, gs_a)
    start_g(1, buf_b, gs_b)

    def body(j, carry):
        i0 = 2 * j
        wait_g(buf_a, gs_a)
        start_s(i0, buf_a, ss_a)
        wait_g(buf_b, gs_b)
        start_s(i0 + 1, buf_b, ss_b)
        wait_s(buf_a, ss_a)
        start_g(i0 + 2, buf_a, gs_a)
        wait_s(buf_b, ss_b)
        start_g(i0 + 3, buf_b, gs_b)
        return carry

    lax.fori_loop(0, _CHM // 2 - 1, body, 0)

    wait_g(buf_a, gs_a)
    start_s(_CHM - 2, buf_a, ss_a)
    wait_g(buf_b, gs_b)
    start_s(_CHM - 1, buf_b, ss_b)
    wait_s(buf_a, ss_a)
    wait_s(buf_b, ss_b)

    plsc.subcore_barrier()
    for t in range(_NZ):
        sl = pl.ds(sid * _RPT + t * _ZR, _ZR)
        pltpu.sync_copy(acc_sp.at[sl], buf_a.at[pl.ds(0, _ZR)])
        pltpu.sync_copy(buf_a.at[pl.ds(0, _ZR)], acc_out.at[cid, sl])


def _prep_body(deg_ref, x_ref, w_ref, y_ref, dis_ref):
    deg = deg_ref[0][:_N, 0:1] + deg_ref[1][:_N, 0:1] + 1.0
    dis = lax.rsqrt(deg)
    xw = jnp.dot(x_ref[...], w_ref[...], preferred_element_type=jnp.float32)
    y_ref[...] = xw * dis
    dis_ref[...] = dis


_prep = pl.pallas_call(
    _prep_body,
    out_shape=(
        jax.ShapeDtypeStruct((_N, _D), jnp.float32),
        jax.ShapeDtypeStruct((_N, 1), jnp.float32),
    ),
)


def _out_body(acc_ref, y_ref, dis_ref, b_ref, out_ref):
    out_ref[...] = (acc_ref[0][:_N] + acc_ref[1][:_N] + y_ref[...]) * dis_ref[...] + b_ref[...]


_outk = pl.pallas_call(
    _out_body,
    out_shape=jax.ShapeDtypeStruct((_N, _D), jnp.float32),
)


@jax.jit
def _run(x, edge_index, W, b):
    pad = _IR * _IC - _EPW
    src = edge_index[0].reshape(_NW, _EPW)
    dst = edge_index[1].reshape(_NW, _EPW)
    src = jnp.pad(src, ((0, 0), (0, pad))).reshape(_NW, _IR, _IC)
    dst = jnp.pad(dst, ((0, 0), (0, pad)), constant_values=_N).reshape(_NW, _IR, _IC)
    degp = _deg_kernel(dst)
    y, dis = _prep(degp, x, W)
    accp = _msg_kernel(y, src, dst)
    return _outk(accp, y, dis, b.reshape(1, _D))


def kernel(x, edge_index, W, b):
    return _run(x, edge_index, W, b)
